# baseline (speedup 1.0000x reference)
import functools

import jax
import jax.numpy as jnp
from jax import lax
from jax.experimental import pallas as pl
from jax.experimental.pallas import tpu as pltpu
from jax.experimental.pallas import tpu_sc as plsc

N = 10000
F = 128
E = 320000
C = 80
ROWS = E // C
NW = 32
TPW = ROWS // NW
NB = 5
B = TPW // NB
STRIPE = 624
TAIL = N - 16 * STRIPE
LG = C // 16


def _sc_agg(srcB, dstB, wB, x, zrows, zvec):
    mesh = plsc.VectorSubcoreMesh(core_axis_name="c", subcore_axis_name="s")

    @functools.partial(
        pl.kernel,
        mesh=mesh,
        compiler_params=pltpu.CompilerParams(needs_layout_passes=False),
        out_type=jax.ShapeDtypeStruct((2, N, F), jnp.float32),
        scratch_types=[
            pltpu.VMEM((B, C), jnp.int32),
            pltpu.VMEM((B, C), jnp.int32),
            pltpu.VMEM((B, C), jnp.float32),
            pltpu.VMEM((B, C), jnp.float32),
            pltpu.VMEM((N,), jnp.float32),
            pltpu.VMEM((C, F), jnp.float32),
            pltpu.VMEM_SHARED((N,), jnp.float32),
            pltpu.VMEM_SHARED((N, F), jnp.float32),
            pltpu.SemaphoreType.DMA,
        ],
    )
    def agg(srcB, dstB, wB, x, zrows, zvec, out,
            sblk, dblk, wblk, nblk, dinv, rows, sdeg, sacc, sem):
        cid = lax.axis_index("c")
        sid = lax.axis_index("s")
        wid = cid * 16 + sid

        pltpu.sync_copy(zrows.at[pl.ds(sid * STRIPE, STRIPE), :],
                        sacc.at[pl.ds(sid * STRIPE, STRIPE), :])

        @pl.when(sid == 15)
        def _():
            pltpu.sync_copy(zrows.at[pl.ds(16 * STRIPE, TAIL), :],
                            sacc.at[pl.ds(16 * STRIPE, TAIL), :])

        @pl.when(sid == 0)
        def _():
            pltpu.sync_copy(zvec, sdeg)

        plsc.subcore_barrier()

        def mask_body(i, _):
            r = i // LG
            l = (i % LG) * 16
            s16 = sblk[r, pl.ds(l, 16)]
            d16 = dblk[r, pl.ds(l, 16)]
            w16 = wblk[r, pl.ds(l, 16)]
            wblk[r, pl.ds(l, 16)] = jnp.where(s16 == d16, 0.0, w16)
            return 0

        def deg_row(r, _):
            pltpu.sync_copy(wblk.at[r], sdeg.at[sblk.at[r]], add=True)
            return 0

        def deg_block(hb, _):
            h = hb // NB
            b = hb % NB
            pltpu.sync_copy(srcB.at[2 * sid + h, b], sblk)
            pltpu.sync_copy(dstB.at[2 * sid + h, b], dblk)
            pltpu.sync_copy(wB.at[2 * sid + h, b], wblk)
            lax.fori_loop(0, B * LG, mask_body, 0)
            lax.fori_loop(0, B, deg_row, 0)
            return 0

        with jax.named_scope("ph1_deg"):
            lax.fori_loop(0, 2 * NB, deg_block, 0)
        plsc.subcore_barrier()

        pltpu.sync_copy(sdeg, dinv)
        magic = jnp.full((16,), 0x5F3759DF, jnp.int32)

        def rsq_body(i, _):
            d = dinv[pl.ds(i * 16, 16)]
            yi = magic - lax.shift_right_logical(
                lax.bitcast_convert_type(d, jnp.int32), 1)
            y = lax.bitcast_convert_type(yi, jnp.float32)
            y = y * (1.5 - 0.5 * d * y * y)
            y = y * (1.5 - 0.5 * d * y * y)
            y = y * (1.5 - 0.5 * d * y * y)
            dinv[pl.ds(i * 16, 16)] = jnp.where(d > 0.0, y, 0.0)
            return 0

        with jax.named_scope("ph2_rsq"):
            lax.fori_loop(0, N // 16, rsq_body, 0)

        def norm_body(i, _):
            r = i // LG
            l = (i % LG) * 16
            s16 = sblk[r, pl.ds(l, 16)]
            d16 = dblk[r, pl.ds(l, 16)]
            w16 = wblk[r, pl.ds(l, 16)]
            wm = jnp.where(s16 == d16, 0.0, w16)
            gs = plsc.load_gather(dinv, [s16])
            gd = plsc.load_gather(dinv, [d16])
            nblk[r, pl.ds(l, 16)] = -(gs * wm * gd)
            return 0

        def chunk_body(rr, _):
            pltpu.async_copy(x.at[sblk.at[rr]], rows, sem).wait()

            def group_body(g, _):
                nv16 = nblk[rr, pl.ds(g * 16, 16)]
                for j in range(16):
                    e = g * 16 + j
                    bc = jnp.full((16,), nv16[j], jnp.float32)
                    for k in range(8):
                        rows[e, pl.ds(k * 16, 16)] = (
                            rows[e, pl.ds(k * 16, 16)] * bc)
                return 0

            lax.fori_loop(0, LG, group_body, 0)
            pltpu.sync_copy(rows, sacc.at[dblk.at[rr]], add=True)
            return 0

        def msg_block(b, _):
            pltpu.sync_copy(srcB.at[wid, b], sblk)
            pltpu.sync_copy(dstB.at[wid, b], dblk)
            pltpu.sync_copy(wB.at[wid, b], wblk)
            lax.fori_loop(0, B * LG, norm_body, 0)
            lax.fori_loop(0, B, chunk_body, 0)
            return 0

        with jax.named_scope("ph3_msg"):
            lax.fori_loop(0, NB, msg_block, 0)
        plsc.subcore_barrier()

        pltpu.sync_copy(sacc.at[pl.ds(sid * STRIPE, STRIPE), :],
                        out.at[cid, pl.ds(sid * STRIPE, STRIPE), :])

        @pl.when(sid == 15)
        def _():
            pltpu.sync_copy(sacc.at[pl.ds(16 * STRIPE, TAIL), :],
                            out.at[cid, pl.ds(16 * STRIPE, TAIL), :])

    return agg(srcB, dstB, wB, x, zrows, zvec)


def _tc_body(x_ref, p0_ref, p1_ref, wzh_ref, bzh_ref, wlt_ref, blin_ref, o_ref):
    t = p0_ref[...] + p1_ref[...]
    xx = x_ref[...]
    logits = jnp.dot(xx, wzh_ref[0:F, :], preferred_element_type=jnp.float32)
    logits = logits + jnp.dot(t, wzh_ref[F:2 * F, :],
                              preferred_element_type=jnp.float32)
    logits = logits + bzh_ref[...]
    z = jax.nn.sigmoid(logits[:, 0:F])
    ht = jnp.tanh(logits[:, F:2 * F])
    h = jnp.maximum((1.0 - z) * ht, 0.0)
    o_ref[...] = jnp.dot(h, wlt_ref[...],
                         preferred_element_type=jnp.float32) + blin_ref[...]


def _tc_dense(x, p0, p1, wzh, bzh, wlt, blin):
    R = 2000
    grid = (N // R,)
    return pl.pallas_call(
        _tc_body,
        grid=grid,
        in_specs=[
            pl.BlockSpec((R, F), lambda i: (i, 0)),
            pl.BlockSpec((R, F), lambda i: (i, 0)),
            pl.BlockSpec((R, F), lambda i: (i, 0)),
            pl.BlockSpec((2 * F, 2 * F), lambda i: (0, 0)),
            pl.BlockSpec((1, 2 * F), lambda i: (0, 0)),
            pl.BlockSpec((F, F), lambda i: (0, 0)),
            pl.BlockSpec((1, F), lambda i: (0, 0)),
        ],
        out_specs=pl.BlockSpec((R, F), lambda i: (i, 0)),
        out_shape=jax.ShapeDtypeStruct((N, F), jnp.float32),
    )(x, p0, p1, wzh, bzh, wlt, blin)


def kernel(x, edge_index, edge_weight, W_xz, b_xz, W_hz, b_hz, W_xr, b_xr,
           W_hr, b_hr, W_xh, b_xh, W_hh, b_hh, W_lin, b_lin):
    src = edge_index[0].astype(jnp.int32)
    dst = edge_index[1].astype(jnp.int32)
    w = edge_weight.astype(jnp.float32)
    srcB = src.reshape(NW, NB, B, C)
    dstB = dst.reshape(NW, NB, B, C)
    wB = w.reshape(NW, NB, B, C)
    zrows = jnp.zeros((N, F), jnp.float32)
    zvec = jnp.zeros((N,), jnp.float32)
    tx1p = _sc_agg(srcB, dstB, wB, x, zrows, zvec)

    wzh = jnp.concatenate([
        jnp.concatenate([W_xz[0], W_xh[0]], axis=1),
        jnp.concatenate([W_xz[1], W_xh[1]], axis=1),
    ], axis=0)
    bzh = jnp.concatenate([b_xz + b_hz, b_xh + b_hh]).reshape(1, 2 * F)
    wlt = W_lin.T
    blin = b_lin.reshape(1, F)
    return _tc_dense(x, tx1p[0], tx1p[1], wzh, bzh, wlt, blin)


# speedup vs baseline: 1.4742x; 1.4742x over previous
import functools

import jax
import jax.numpy as jnp
from jax import lax
from jax.experimental import pallas as pl
from jax.experimental.pallas import tpu as pltpu
from jax.experimental.pallas import tpu_sc as plsc

N = 10000
F = 128
E = 320000
C = 80
ROWS = E // C
NW = 32
TPW = ROWS // NW
NB = 5
B = TPW // NB
STRIPE = 624
TAIL = N - 16 * STRIPE
LG = C // 16


def _sc_agg(srcB, dstB, wB, x, zrows, zvec):
    mesh = plsc.VectorSubcoreMesh(core_axis_name="c", subcore_axis_name="s")

    @functools.partial(
        pl.kernel,
        mesh=mesh,
        compiler_params=pltpu.CompilerParams(needs_layout_passes=False),
        out_type=jax.ShapeDtypeStruct((2, N, F), jnp.float32),
        scratch_types=[
            pltpu.VMEM((B, C), jnp.int32),
            pltpu.VMEM((B, C), jnp.int32),
            pltpu.VMEM((B, C), jnp.float32),
            pltpu.VMEM((B, C), jnp.float32),
            pltpu.VMEM((N,), jnp.float32),
            pltpu.VMEM((C, F), jnp.float32),
            pltpu.VMEM((C, F), jnp.float32),
            pltpu.VMEM_SHARED((N,), jnp.float32),
            pltpu.VMEM_SHARED((N, F), jnp.float32),
            pltpu.SemaphoreType.DMA,
            pltpu.SemaphoreType.DMA,
        ],
    )
    def agg(srcB, dstB, wB, x, zrows, zvec, out,
            sblk, dblk, wblk, nblk, dinv, rowsA, rowsB, sdeg, sacc,
            semA, semB):
        cid = lax.axis_index("c")
        sid = lax.axis_index("s")
        wid = cid * 16 + sid

        pltpu.sync_copy(zrows.at[pl.ds(sid * STRIPE, STRIPE), :],
                        sacc.at[pl.ds(sid * STRIPE, STRIPE), :])

        @pl.when(sid == 15)
        def _():
            pltpu.sync_copy(zrows.at[pl.ds(16 * STRIPE, TAIL), :],
                            sacc.at[pl.ds(16 * STRIPE, TAIL), :])

        @pl.when(sid == 0)
        def _():
            pltpu.sync_copy(zvec, sdeg)

        plsc.subcore_barrier()

        def mask_body(i, _):
            r = i // LG
            l = (i % LG) * 16
            s16 = sblk[r, pl.ds(l, 16)]
            d16 = dblk[r, pl.ds(l, 16)]
            w16 = wblk[r, pl.ds(l, 16)]
            wblk[r, pl.ds(l, 16)] = jnp.where(s16 == d16, 0.0, w16)
            return 0

        def deg_block(hb, _):
            h = hb // NB
            b = hb % NB
            pltpu.sync_copy(srcB.at[2 * sid + h, b], sblk)
            pltpu.sync_copy(dstB.at[2 * sid + h, b], dblk)
            pltpu.sync_copy(wB.at[2 * sid + h, b], wblk)
            lax.fori_loop(0, B * LG, mask_body, 0)

            def deg_fire(r, _):
                pltpu.async_copy(wblk.at[r], sdeg.at[sblk.at[r]], semA,
                                 add=True)
                return 0

            def deg_drain(r, _):
                pltpu.make_async_copy(wblk.at[r], sdeg.at[sblk.at[r]],
                                      semA).wait()
                return 0

            lax.fori_loop(0, B, deg_fire, 0)
            lax.fori_loop(0, B, deg_drain, 0)
            return 0

        with jax.named_scope("ph1_deg"):
            lax.fori_loop(0, 2 * NB, deg_block, 0)
        plsc.subcore_barrier()

        pltpu.sync_copy(sdeg, dinv)
        magic = jnp.full((16,), 0x5F3759DF, jnp.int32)

        def rsq_body(i, _):
            d = dinv[pl.ds(i * 16, 16)]
            yi = magic - lax.shift_right_logical(
                lax.bitcast_convert_type(d, jnp.int32), 1)
            y = lax.bitcast_convert_type(yi, jnp.float32)
            y = y * (1.5 - 0.5 * d * y * y)
            y = y * (1.5 - 0.5 * d * y * y)
            y = y * (1.5 - 0.5 * d * y * y)
            dinv[pl.ds(i * 16, 16)] = jnp.where(d > 0.0, y, 0.0)
            return 0

        with jax.named_scope("ph2_rsq"):
            lax.fori_loop(0, N // 16, rsq_body, 0)

        def norm_body(i, _):
            r = i // LG
            l = (i % LG) * 16
            s16 = sblk[r, pl.ds(l, 16)]
            d16 = dblk[r, pl.ds(l, 16)]
            w16 = wblk[r, pl.ds(l, 16)]
            wm = jnp.where(s16 == d16, 0.0, w16)
            gs = plsc.load_gather(dinv, [s16])
            gd = plsc.load_gather(dinv, [d16])
            nblk[r, pl.ds(l, 16)] = -(gs * wm * gd)
            return 0

        def scale_scatter(rows, rr):
            def group_body(g, _):
                nv16 = nblk[rr, pl.ds(g * 16, 16)]
                for j in range(16):
                    e = g * 16 + j
                    bc = jnp.full((16,), nv16[j], jnp.float32)
                    for k in range(8):
                        rows[e, pl.ds(k * 16, 16)] = (
                            rows[e, pl.ds(k * 16, 16)] * bc)
                return 0

            lax.fori_loop(0, LG, group_body, 0)
            pltpu.sync_copy(rows, sacc.at[dblk.at[rr]], add=True)

        def msg_block(b, _):
            pltpu.sync_copy(srcB.at[wid, b], sblk)
            pltpu.sync_copy(dstB.at[wid, b], dblk)
            pltpu.sync_copy(wB.at[wid, b], wblk)
            lax.fori_loop(0, B * LG, norm_body, 0)

            gA = pltpu.async_copy(x.at[sblk.at[0]], rowsA, semA)
            gB = pltpu.async_copy(x.at[sblk.at[1]], rowsB, semB)

            def pair_body(p, _):
                gA.wait()
                scale_scatter(rowsA, 2 * p)
                pltpu.async_copy(x.at[sblk.at[2 * p + 2]], rowsA, semA)
                gB.wait()
                scale_scatter(rowsB, 2 * p + 1)

                @pl.when(p < (B - 3) // 2)
                def _():
                    pltpu.async_copy(x.at[sblk.at[2 * p + 3]], rowsB, semB)

                return 0

            lax.fori_loop(0, (B - 1) // 2, pair_body, 0)
            gA.wait()
            scale_scatter(rowsA, B - 1)
            return 0

        with jax.named_scope("ph3_msg"):
            lax.fori_loop(0, NB, msg_block, 0)
        plsc.subcore_barrier()

        pltpu.sync_copy(sacc.at[pl.ds(sid * STRIPE, STRIPE), :],
                        out.at[cid, pl.ds(sid * STRIPE, STRIPE), :])

        @pl.when(sid == 15)
        def _():
            pltpu.sync_copy(sacc.at[pl.ds(16 * STRIPE, TAIL), :],
                            out.at[cid, pl.ds(16 * STRIPE, TAIL), :])

    return agg(srcB, dstB, wB, x, zrows, zvec)


def _tc_body(x_ref, p0_ref, p1_ref, wzh_ref, bzh_ref, wlt_ref, blin_ref, o_ref):
    t = p0_ref[...] + p1_ref[...]
    xx = x_ref[...]
    logits = jnp.dot(xx, wzh_ref[0:F, :], preferred_element_type=jnp.float32)
    logits = logits + jnp.dot(t, wzh_ref[F:2 * F, :],
                              preferred_element_type=jnp.float32)
    logits = logits + bzh_ref[...]
    z = jax.nn.sigmoid(logits[:, 0:F])
    ht = jnp.tanh(logits[:, F:2 * F])
    h = jnp.maximum((1.0 - z) * ht, 0.0)
    o_ref[...] = jnp.dot(h, wlt_ref[...],
                         preferred_element_type=jnp.float32) + blin_ref[...]


def _tc_dense(x, p0, p1, wzh, bzh, wlt, blin):
    R = 2000
    grid = (N // R,)
    return pl.pallas_call(
        _tc_body,
        grid=grid,
        in_specs=[
            pl.BlockSpec((R, F), lambda i: (i, 0)),
            pl.BlockSpec((R, F), lambda i: (i, 0)),
            pl.BlockSpec((R, F), lambda i: (i, 0)),
            pl.BlockSpec((2 * F, 2 * F), lambda i: (0, 0)),
            pl.BlockSpec((1, 2 * F), lambda i: (0, 0)),
            pl.BlockSpec((F, F), lambda i: (0, 0)),
            pl.BlockSpec((1, F), lambda i: (0, 0)),
        ],
        out_specs=pl.BlockSpec((R, F), lambda i: (i, 0)),
        out_shape=jax.ShapeDtypeStruct((N, F), jnp.float32),
    )(x, p0, p1, wzh, bzh, wlt, blin)


def kernel(x, edge_index, edge_weight, W_xz, b_xz, W_hz, b_hz, W_xr, b_xr,
           W_hr, b_hr, W_xh, b_xh, W_hh, b_hh, W_lin, b_lin):
    src = edge_index[0].astype(jnp.int32)
    dst = edge_index[1].astype(jnp.int32)
    w = edge_weight.astype(jnp.float32)
    srcB = src.reshape(NW, NB, B, C)
    dstB = dst.reshape(NW, NB, B, C)
    wB = w.reshape(NW, NB, B, C)
    zrows = jnp.zeros((N, F), jnp.float32)
    zvec = jnp.zeros((N,), jnp.float32)
    tx1p = _sc_agg(srcB, dstB, wB, x, zrows, zvec)

    wzh = jnp.concatenate([
        jnp.concatenate([W_xz[0], W_xh[0]], axis=1),
        jnp.concatenate([W_xz[1], W_xh[1]], axis=1),
    ], axis=0)
    bzh = jnp.concatenate([b_xz + b_hz, b_xh + b_hh]).reshape(1, 2 * F)
    wlt = W_lin.T
    blin = b_lin.reshape(1, F)
    return _tc_dense(x, tx1p[0], tx1p[1], wzh, bzh, wlt, blin)


# speedup vs baseline: 1.6297x; 1.1055x over previous
import functools

import jax
import jax.numpy as jnp
from jax import lax
from jax.experimental import pallas as pl
from jax.experimental.pallas import tpu as pltpu
from jax.experimental.pallas import tpu_sc as plsc

N = 10000
F = 128
E = 320000
C = 80
ROWS = E // C
NW = 32
TPW = ROWS // NW
NB = 5
B = TPW // NB
STRIPE = 624
TAIL = N - 16 * STRIPE
LG = C // 16


def _sc_agg(e4, w4, x):
    mesh = plsc.VectorSubcoreMesh(core_axis_name="c", subcore_axis_name="s")

    @functools.partial(
        pl.kernel,
        mesh=mesh,
        compiler_params=pltpu.CompilerParams(needs_layout_passes=False),
        out_type=jax.ShapeDtypeStruct((2, N, F), jnp.float32),
        scratch_types=[
            pltpu.VMEM((B, C), jnp.int32),
            pltpu.VMEM((B, C), jnp.int32),
            pltpu.VMEM((B, C), jnp.float32),
            pltpu.VMEM((B, C), jnp.float32),
            pltpu.VMEM((N,), jnp.float32),
            pltpu.VMEM((C, F), jnp.float32),
            pltpu.VMEM((C, F), jnp.float32),
            pltpu.VMEM_SHARED((N,), jnp.float32),
            pltpu.VMEM_SHARED((N,), jnp.float32),
            pltpu.VMEM_SHARED((N, F), jnp.float32),
            pltpu.SemaphoreType.DMA,
            pltpu.SemaphoreType.DMA,
        ],
    )
    def agg(e4, w4, x, out,
            sblk, dblk, wblk, nblk, dinv, rowsA, rowsB, sdeg, sdinv,
            sacc, semA, semB):
        cid = lax.axis_index("c")
        sid = lax.axis_index("s")
        wid = cid * 16 + sid

        z16 = jnp.zeros((16,), jnp.float32)

        def zrow_body(r, _):
            for k in range(8):
                rowsA[r, pl.ds(k * 16, 16)] = z16
            return 0

        lax.fori_loop(0, C, zrow_body, 0)

        def zdinv_body(i, _):
            dinv[pl.ds(i * 16, 16)] = z16
            return 0

        lax.fori_loop(0, N // 16, zdinv_body, 0)

        for j in range(STRIPE // C):
            pltpu.sync_copy(rowsA, sacc.at[pl.ds(sid * STRIPE + j * C, C), :])
        pltpu.sync_copy(rowsA.at[pl.ds(0, STRIPE - (STRIPE // C) * C), :],
                        sacc.at[pl.ds(sid * STRIPE + (STRIPE // C) * C,
                                      STRIPE - (STRIPE // C) * C), :])
        pltpu.sync_copy(dinv.at[pl.ds(0, STRIPE)],
                        sdeg.at[pl.ds(sid * STRIPE, STRIPE)])

        @pl.when(sid == 15)
        def _():
            pltpu.sync_copy(rowsA.at[pl.ds(0, TAIL), :],
                            sacc.at[pl.ds(16 * STRIPE, TAIL), :])
            pltpu.sync_copy(dinv.at[pl.ds(0, TAIL)],
                            sdeg.at[pl.ds(16 * STRIPE, TAIL)])

        plsc.subcore_barrier()

        def mask_body(i, _):
            r = i // LG
            l = (i % LG) * 16
            s16 = sblk[r, pl.ds(l, 16)]
            d16 = dblk[r, pl.ds(l, 16)]
            w16 = wblk[r, pl.ds(l, 16)]
            wblk[r, pl.ds(l, 16)] = jnp.where(s16 == d16, 0.0, w16)
            return 0

        def deg_block(hb, _):
            h = hb // NB
            b = hb % NB
            pltpu.sync_copy(e4.at[0, 2 * sid + h, b], sblk)
            pltpu.sync_copy(e4.at[1, 2 * sid + h, b], dblk)
            pltpu.sync_copy(w4.at[2 * sid + h, b], wblk)
            lax.fori_loop(0, B * LG, mask_body, 0)

            def deg_fire(r, _):
                pltpu.async_copy(wblk.at[r], sdeg.at[sblk.at[r]], semA,
                                 add=True)
                return 0

            def deg_drain(r, _):
                pltpu.make_async_copy(wblk.at[r], sdeg.at[sblk.at[r]],
                                      semA).wait()
                return 0

            lax.fori_loop(0, B, deg_fire, 0)
            lax.fori_loop(0, B, deg_drain, 0)
            return 0

        with jax.named_scope("ph1_deg"):
            lax.fori_loop(0, 2 * NB, deg_block, 0)
        plsc.subcore_barrier()

        magic = jnp.full((16,), 0x5F3759DF, jnp.int32)

        def rsq_body(i, _):
            d = dinv[pl.ds(i * 16, 16)]
            yi = magic - lax.shift_right_logical(
                lax.bitcast_convert_type(d, jnp.int32), 1)
            y = lax.bitcast_convert_type(yi, jnp.float32)
            y = y * (1.5 - 0.5 * d * y * y)
            y = y * (1.5 - 0.5 * d * y * y)
            y = y * (1.5 - 0.5 * d * y * y)
            dinv[pl.ds(i * 16, 16)] = jnp.where(d > 0.0, y, 0.0)
            return 0

        with jax.named_scope("ph2_rsq"):
            pltpu.sync_copy(sdeg.at[pl.ds(sid * STRIPE, STRIPE)],
                            dinv.at[pl.ds(0, STRIPE)])

            @pl.when(sid == 15)
            def _():
                pltpu.sync_copy(sdeg.at[pl.ds(16 * STRIPE, TAIL)],
                                dinv.at[pl.ds(STRIPE, TAIL)])

            lax.fori_loop(0, STRIPE // 16, rsq_body, 0)

            @pl.when(sid == 15)
            def _():
                lax.fori_loop(STRIPE // 16, (STRIPE + TAIL) // 16,
                              rsq_body, 0)

            pltpu.sync_copy(dinv.at[pl.ds(0, STRIPE)],
                            sdinv.at[pl.ds(sid * STRIPE, STRIPE)])

            @pl.when(sid == 15)
            def _():
                pltpu.sync_copy(dinv.at[pl.ds(STRIPE, TAIL)],
                                sdinv.at[pl.ds(16 * STRIPE, TAIL)])

            plsc.subcore_barrier()
            pltpu.sync_copy(sdinv, dinv)

        def norm_body(i, _):
            r = i // LG
            l = (i % LG) * 16
            s16 = sblk[r, pl.ds(l, 16)]
            d16 = dblk[r, pl.ds(l, 16)]
            w16 = wblk[r, pl.ds(l, 16)]
            wm = jnp.where(s16 == d16, 0.0, w16)
            gs = plsc.load_gather(dinv, [s16])
            gd = plsc.load_gather(dinv, [d16])
            nblk[r, pl.ds(l, 16)] = -(gs * wm * gd)
            return 0

        def scale_scatter(rows, rr):
            def group_body(g, _):
                nv16 = nblk[rr, pl.ds(g * 16, 16)]
                for j in range(16):
                    e = g * 16 + j
                    bc = jnp.full((16,), nv16[j], jnp.float32)
                    for k in range(8):
                        rows[e, pl.ds(k * 16, 16)] = (
                            rows[e, pl.ds(k * 16, 16)] * bc)
                return 0

            lax.fori_loop(0, LG, group_body, 0)
            pltpu.sync_copy(rows, sacc.at[dblk.at[rr]], add=True)

        def msg_block(b, _):
            pltpu.sync_copy(e4.at[0, wid, b], sblk)
            pltpu.sync_copy(e4.at[1, wid, b], dblk)
            pltpu.sync_copy(w4.at[wid, b], wblk)
            lax.fori_loop(0, B * LG, norm_body, 0)

            gA = pltpu.async_copy(x.at[sblk.at[0]], rowsA, semA)
            gB = pltpu.async_copy(x.at[sblk.at[1]], rowsB, semB)

            def pair_body(p, _):
                gA.wait()
                scale_scatter(rowsA, 2 * p)
                pltpu.async_copy(x.at[sblk.at[2 * p + 2]], rowsA, semA)
                gB.wait()
                scale_scatter(rowsB, 2 * p + 1)

                @pl.when(p < (B - 3) // 2)
                def _():
                    pltpu.async_copy(x.at[sblk.at[2 * p + 3]], rowsB, semB)

                return 0

            lax.fori_loop(0, (B - 1) // 2, pair_body, 0)
            gA.wait()
            scale_scatter(rowsA, B - 1)
            return 0

        with jax.named_scope("ph3_msg"):
            lax.fori_loop(0, NB, msg_block, 0)
        plsc.subcore_barrier()

        pltpu.sync_copy(sacc.at[pl.ds(sid * STRIPE, STRIPE), :],
                        out.at[cid, pl.ds(sid * STRIPE, STRIPE), :])

        @pl.when(sid == 15)
        def _():
            pltpu.sync_copy(sacc.at[pl.ds(16 * STRIPE, TAIL), :],
                            out.at[cid, pl.ds(16 * STRIPE, TAIL), :])

    return agg(e4, w4, x)


def _tc_body(x_ref, p0_ref, p1_ref, wzh_ref, bzh_ref, wlt_ref, blin_ref, o_ref):
    t = p0_ref[...] + p1_ref[...]
    xx = x_ref[...]
    logits = jnp.dot(xx, wzh_ref[0:F, :], preferred_element_type=jnp.float32)
    logits = logits + jnp.dot(t, wzh_ref[F:2 * F, :],
                              preferred_element_type=jnp.float32)
    logits = logits + bzh_ref[...]
    z = jax.nn.sigmoid(logits[:, 0:F])
    ht = jnp.tanh(logits[:, F:2 * F])
    h = jnp.maximum((1.0 - z) * ht, 0.0)
    o_ref[...] = jnp.dot(h, wlt_ref[...],
                         preferred_element_type=jnp.float32) + blin_ref[...]


def _tc_dense(x, p0, p1, wzh, bzh, wlt, blin):
    R = 2000
    grid = (N // R,)
    return pl.pallas_call(
        _tc_body,
        grid=grid,
        in_specs=[
            pl.BlockSpec((R, F), lambda i: (i, 0)),
            pl.BlockSpec((R, F), lambda i: (i, 0)),
            pl.BlockSpec((R, F), lambda i: (i, 0)),
            pl.BlockSpec((2 * F, 2 * F), lambda i: (0, 0)),
            pl.BlockSpec((1, 2 * F), lambda i: (0, 0)),
            pl.BlockSpec((F, F), lambda i: (0, 0)),
            pl.BlockSpec((1, F), lambda i: (0, 0)),
        ],
        out_specs=pl.BlockSpec((R, F), lambda i: (i, 0)),
        out_shape=jax.ShapeDtypeStruct((N, F), jnp.float32),
    )(x, p0, p1, wzh, bzh, wlt, blin)


def kernel(x, edge_index, edge_weight, W_xz, b_xz, W_hz, b_hz, W_xr, b_xr,
           W_hr, b_hr, W_xh, b_xh, W_hh, b_hh, W_lin, b_lin):
    e4 = edge_index.astype(jnp.int32).reshape(2, NW, NB, B, C)
    w4 = edge_weight.astype(jnp.float32).reshape(NW, NB, B, C)
    tx1p = _sc_agg(e4, w4, x)

    wzh = jnp.concatenate([
        jnp.concatenate([W_xz[0], W_xh[0]], axis=1),
        jnp.concatenate([W_xz[1], W_xh[1]], axis=1),
    ], axis=0)
    bzh = jnp.concatenate([b_xz + b_hz, b_xh + b_hh]).reshape(1, 2 * F)
    wlt = W_lin.T
    blin = b_lin.reshape(1, F)
    return _tc_dense(x, tx1p[0], tx1p[1], wzh, bzh, wlt, blin)
